# baseline (speedup 1.0000x reference)
import functools

import jax
import jax.numpy as jnp
from jax import lax
from jax.experimental import pallas as pl
from jax.experimental.pallas import tpu as pltpu
from jax.experimental.pallas import tpu_sc as plsc

N = 10000
E = 320000
D = 128
NPAD = 10240
TRASH = NPAD - N
NT = 16
CHUNK = 88
GSZ = 8
NCHUNK = 232
NGRP = NCHUNK // GSZ
EPAD = NT * NCHUNK * CHUNK
ZROWS = NPAD // NT
BLK = 1024
GRID = NPAD // BLK

_mesh = plsc.VectorSubcoreMesh(core_axis_name="c", subcore_axis_name="s")


CH_D = 128
NCH_D = 160


def _deg_body(z1, s1, d1, s2, d2, s3, d3, s4, d4,
              os1, od1, os2, od2, os3, od3, os4, od4,
              sbufs, dbufs, ones_v, hs_spm, hd_spm):
    c = lax.axis_index("c")
    s = lax.axis_index("s")
    for j in range(CH_D // 16):
        ones_v[pl.ds(j * 16, 16)] = jnp.ones((16,), jnp.float32)

    def do_graph(si, di, os_hbm, od_hbm):
        pltpu.sync_copy(z1, hs_spm.at[pl.ds(s * ZROWS, ZROWS)])
        pltpu.sync_copy(z1, hd_spm.at[pl.ds(s * ZROWS, ZROWS)])
        pltpu.sync_copy(si.at[s], sbufs)
        pltpu.sync_copy(di.at[s], dbufs)
        plsc.subcore_barrier()

        @pl.loop(0, NCH_D)
        def _(i):
            pltpu.sync_copy(ones_v, hs_spm.at[sbufs.at[i]], add=True)
            pltpu.sync_copy(ones_v, hd_spm.at[dbufs.at[i]], add=True)

        plsc.subcore_barrier()
        pltpu.sync_copy(hs_spm.at[pl.ds(s * ZROWS, ZROWS)],
                        os_hbm.at[pl.ds(s * ZROWS, ZROWS)])
        pltpu.sync_copy(hd_spm.at[pl.ds(s * ZROWS, ZROWS)],
                        od_hbm.at[pl.ds(s * ZROWS, ZROWS)])
        plsc.subcore_barrier()

    @pl.when(c == 0)
    def _():
        do_graph(s1, d1, os1, od1)
        do_graph(s2, d2, os2, od2)

    @pl.when(c == 1)
    def _():
        do_graph(s3, d3, os3, od3)
        do_graph(s4, d4, os4, od4)


_deg_call = functools.partial(
    pl.kernel,
    out_type=[jax.ShapeDtypeStruct((NPAD,), jnp.float32)] * 8,
    mesh=_mesh,
    scratch_types=[
        pltpu.VMEM((NCH_D, CH_D), jnp.int32),
        pltpu.VMEM((NCH_D, CH_D), jnp.int32),
        pltpu.VMEM((CH_D,), jnp.float32),
        pltpu.VMEM_SHARED((NPAD,), jnp.float32),
        pltpu.VMEM_SHARED((NPAD,), jnp.float32),
    ],
)(_deg_body)


def _agg_body(z, h1, s1, d1, h2, s2, d2, h3, s3, d3, h4, s4, d4,
              o1, o2, o3, o4, sbA, dbA, sbB, dbB,
              rows0, rows1, rows2, rows3,
              gs0, gs1, gs2, gs3, ss0, ss1, ss2, ss3, aspm):
    c = lax.axis_index("c")
    s = lax.axis_index("s")
    rows = (rows0, rows1, rows2, rows3)
    gsem = (gs0, gs1, gs2, gs3)
    ssem = (ss0, ss1, ss2, ss3)

    def do_graph(h, si, di, o):
        def load_grp(g, sb, db):
            pltpu.sync_copy(si.at[s, pl.ds(g * GSZ, GSZ)], sb)
            pltpu.sync_copy(di.at[s, pl.ds(g * GSZ, GSZ)], db)

        def step(j, sb, db, sbn, dbn, wait_ss=True, cross=True):
            p = j % 4
            pltpu.make_async_copy(h.at[sb.at[j]], rows[p], gsem[p]).wait()
            pltpu.async_copy(rows[p], aspm.at[db.at[j]], ssem[p], add=True)
            q = (j + 2) % 4
            if wait_ss:
                pltpu.make_async_copy(rows[q], aspm.at[db.at[j - 2]],
                                      ssem[q]).wait()
            if cross:
                if j + 2 < GSZ:
                    pltpu.async_copy(h.at[sb.at[j + 2]], rows[q], gsem[q])
                else:
                    pltpu.async_copy(h.at[sbn.at[j - 6]], rows[q], gsem[q])

        load_grp(0, sbA, dbA)
        pltpu.async_copy(h.at[sbA.at[0]], rows[0], gsem[0])
        pltpu.async_copy(h.at[sbA.at[1]], rows[1], gsem[1])
        pltpu.sync_copy(z, aspm.at[pl.ds(s * ZROWS, ZROWS)])
        plsc.subcore_barrier()
        step(0, sbA, dbA, sbB, dbB, wait_ss=False)
        load_grp(1, sbB, dbB)
        step(1, sbA, dbA, sbB, dbB, wait_ss=False)
        for j in range(2, GSZ):
            step(j, sbA, dbA, sbB, dbB)

        @pl.loop(1, NGRP - 1, step=2)
        def _(g0):
            step(0, sbB, dbB, sbA, dbA)
            step(1, sbB, dbB, sbA, dbA)
            load_grp(g0 + 1, sbA, dbA)
            for j in range(2, GSZ):
                step(j, sbB, dbB, sbA, dbA)

            step(0, sbA, dbA, sbB, dbB)
            step(1, sbA, dbA, sbB, dbB)

            @pl.when(g0 + 2 < NGRP)
            def _():
                load_grp(g0 + 2, sbB, dbB)

            for j in range(2, GSZ - 2):
                step(j, sbA, dbA, sbB, dbB)
            for j in range(GSZ - 2, GSZ):
                step(j, sbA, dbA, sbB, dbB, cross=False)

                @pl.when(g0 + 2 < NGRP)
                def _():
                    q = (j + 2) % 4
                    pltpu.async_copy(h.at[sbB.at[j - 6]], rows[q], gsem[q])

        for j in range(GSZ - 2, GSZ):
            p = j % 4
            pltpu.make_async_copy(rows[p], aspm.at[dbA.at[j]],
                                  ssem[p]).wait()

        plsc.subcore_barrier()
        pltpu.sync_copy(aspm.at[pl.ds(s * ZROWS, ZROWS)],
                        o.at[pl.ds(s * ZROWS, ZROWS)])
        plsc.subcore_barrier()

    @pl.when(c == 0)
    def _():
        do_graph(h1, s1, d1, o1)
        do_graph(h2, s2, d2, o2)

    @pl.when(c == 1)
    def _():
        do_graph(h3, s3, d3, o3)
        do_graph(h4, s4, d4, o4)


_agg_call = functools.partial(
    pl.kernel,
    out_type=[jax.ShapeDtypeStruct((NPAD, D), jnp.float32)] * 4,
    mesh=_mesh,
    scratch_types=[
        pltpu.VMEM((GSZ, CHUNK), jnp.int32),
        pltpu.VMEM((GSZ, CHUNK), jnp.int32),
        pltpu.VMEM((GSZ, CHUNK), jnp.int32),
        pltpu.VMEM((GSZ, CHUNK), jnp.int32),
        pltpu.VMEM((CHUNK, D), jnp.float32),
        pltpu.VMEM((CHUNK, D), jnp.float32),
        pltpu.VMEM((CHUNK, D), jnp.float32),
        pltpu.VMEM((CHUNK, D), jnp.float32),
        pltpu.SemaphoreType.DMA,
        pltpu.SemaphoreType.DMA,
        pltpu.SemaphoreType.DMA,
        pltpu.SemaphoreType.DMA,
        pltpu.SemaphoreType.DMA,
        pltpu.SemaphoreType.DMA,
        pltpu.SemaphoreType.DMA,
        pltpu.SemaphoreType.DMA,
        pltpu.VMEM_SHARED((NPAD, D), jnp.float32),
    ],
)(_agg_body)


def _prep_body(*refs):
    xs = refs[0:4]
    ods = refs[4:8]
    ids_ = refs[8:12]
    hs = refs[12:16]
    oss = refs[16:20]
    iss = refs[20:24]
    for g in range(4):
        osv = lax.rsqrt(jnp.maximum(ods[g][...], 1.0))
        isv = lax.rsqrt(jnp.maximum(ids_[g][...], 1.0))
        oss[g][...] = osv
        iss[g][...] = isv
        hs[g][...] = xs[g][...] * osv


def _prep(xp, od, idg):
    bx = pl.BlockSpec((BLK, D), lambda i: (i, 0))
    bs = pl.BlockSpec((BLK, 1), lambda i: (i, 0))
    return pl.pallas_call(
        _prep_body,
        grid=(GRID,),
        in_specs=[bx] * 4 + [bs] * 8,
        out_specs=[bx] * 4 + [bs] * 8,
        out_shape=[jax.ShapeDtypeStruct((NPAD, D), jnp.float32)] * 4
        + [jax.ShapeDtypeStruct((NPAD, 1), jnp.float32)] * 8,
    )(*xp, *od, *idg)


def _layer_body(w_ref, b_ref, *refs):
    aggs = refs[0:4]
    iss = refs[4:8]
    oss = refs[8:12]
    outs = refs[12:16]
    w = w_ref[...]
    b = b_ref[...]
    for g in range(4):
        a = aggs[g][...] * iss[g][...]
        h = jnp.maximum(jnp.dot(a, w, preferred_element_type=jnp.float32) + b,
                        0.0)
        outs[g][...] = h * oss[g][...]


def _layer(W, b, aggs, iss, oss):
    bx = pl.BlockSpec((BLK, D), lambda i: (i, 0))
    bs = pl.BlockSpec((BLK, 1), lambda i: (i, 0))
    bw = pl.BlockSpec((D, D), lambda i: (0, 0))
    bb = pl.BlockSpec((1, D), lambda i: (0, 0))
    return pl.pallas_call(
        _layer_body,
        grid=(GRID,),
        in_specs=[bw, bb] + [bx] * 4 + [bs] * 8,
        out_specs=[bx] * 4,
        out_shape=[jax.ShapeDtypeStruct((NPAD, D), jnp.float32)] * 4,
    )(W, b, *aggs, *iss, *oss)


def _fin_body(w_ref, b_ref, *refs):
    aggs = refs[0:4]
    iss = refs[4:8]
    out = refs[8]
    i = pl.program_id(0)

    @pl.when(i == 0)
    def _():
        out[...] = jnp.zeros((1, 1), jnp.float32)

    w = w_ref[...]
    b = b_ref[...]
    rowid = lax.broadcasted_iota(jnp.int32, (BLK, 1), 0) + i * BLK
    mask = rowid < N
    acc = jnp.float32(0.0)
    for g in range(4):
        a = aggs[g][...] * iss[g][...]
        h = jnp.maximum(jnp.dot(a, w, preferred_element_type=jnp.float32) + b,
                        0.0)
        acc += jnp.sum(jnp.where(mask, h, 0.0))
    out[...] = out[...] + jnp.reshape(acc, (1, 1))


def _fin(W, b, aggs, iss):
    bx = pl.BlockSpec((BLK, D), lambda i: (i, 0))
    bs = pl.BlockSpec((BLK, 1), lambda i: (i, 0))
    bw = pl.BlockSpec((D, D), lambda i: (0, 0))
    bb = pl.BlockSpec((1, D), lambda i: (0, 0))
    return pl.pallas_call(
        _fin_body,
        grid=(GRID,),
        in_specs=[bw, bb] + [bx] * 4 + [bs] * 4,
        out_specs=pl.BlockSpec((1, 1), lambda i: (0, 0)),
        out_shape=jax.ShapeDtypeStruct((1, 1), jnp.float32),
    )(W, b, *aggs, *iss)


def _prep_edges(g):
    g = g.astype(jnp.int32)
    pad = N + (jnp.arange(EPAD - E, dtype=jnp.int32) % TRASH)
    src = jnp.concatenate([g[0], pad]).reshape(NT, NCHUNK, CHUNK)
    dst = jnp.concatenate([g[1], pad]).reshape(NT, NCHUNK, CHUNK)
    return src, dst


def kernel(g1, x1, g2, x2, g3, x3, g4, x4, W1, b1, W2, b2, W3, b3):
    srcs, dsts = zip(*(_prep_edges(g) for g in (g1, g2, g3, g4)))

    def _degfmt(a):
        flat = a.reshape(NT, NCHUNK * CHUNK)
        extra = NCH_D * CH_D - NCHUNK * CHUNK
        pad = N + (jnp.arange(extra, dtype=jnp.int32) % TRASH)
        pad = jnp.broadcast_to(pad, (NT, extra))
        return jnp.concatenate([flat, pad], axis=1).reshape(NT, NCH_D, CH_D)

    sf = [_degfmt(a) for a in srcs]
    df = [_degfmt(a) for a in dsts]
    xp = [jnp.pad(x, ((0, TRASH), (0, 0))) for x in (x1, x2, x3, x4)]
    z = jnp.zeros((ZROWS, D), jnp.float32)
    z1 = jnp.zeros((ZROWS,), jnp.float32)

    degs = _deg_call(z1, sf[0], df[0], sf[1], df[1], sf[2], df[2],
                     sf[3], df[3])
    od = [degs[2 * g][:, None] for g in range(4)]
    idg = [degs[2 * g + 1][:, None] for g in range(4)]

    prep = _prep(xp, od, idg)
    hs, oss, iss = prep[0:4], prep[4:8], prep[8:12]

    for W, b in ((W1, b1), (W2, b2)):
        aggs = _agg_call(z, hs[0], srcs[0], dsts[0], hs[1], srcs[1], dsts[1],
                         hs[2], srcs[2], dsts[2], hs[3], srcs[3], dsts[3])
        hs = _layer(W, b.reshape(1, D), aggs, iss, oss)

    aggs = _agg_call(z, hs[0], srcs[0], dsts[0], hs[1], srcs[1], dsts[1],
                     hs[2], srcs[2], dsts[2], hs[3], srcs[3], dsts[3])
    total = _fin(W3, b3.reshape(1, D), aggs, iss)
    return total[0, 0] / jnp.float32(4 * N * D)


# speedup vs baseline: 1.0006x; 1.0006x over previous
import functools

import jax
import jax.numpy as jnp
from jax import lax
from jax.experimental import pallas as pl
from jax.experimental.pallas import tpu as pltpu
from jax.experimental.pallas import tpu_sc as plsc

N = 10000
E = 320000
D = 128
NPAD = 10240
TRASH = NPAD - N
NT = 16
CHUNK = 88
GSZ = 8
NCHUNK = 232
NGRP = NCHUNK // GSZ
EPAD = NT * NCHUNK * CHUNK
ZROWS = NPAD // NT
BLK = 1024
GRID = NPAD // BLK

@functools.cache
def _sc_mesh():
    return plsc.VectorSubcoreMesh(core_axis_name="c", subcore_axis_name="s",
                                  num_cores=2, num_subcores=NT)


CH_D = 128
NCH_D = 160


def _deg_body(z1, s1, d1, s2, d2, s3, d3, s4, d4,
              os1, od1, os2, od2, os3, od3, os4, od4,
              sbufs, dbufs, ones_v, hs_spm, hd_spm):
    c = lax.axis_index("c")
    s = lax.axis_index("s")
    for j in range(CH_D // 16):
        ones_v[pl.ds(j * 16, 16)] = jnp.ones((16,), jnp.float32)

    def do_graph(si, di, os_hbm, od_hbm):
        pltpu.sync_copy(z1, hs_spm.at[pl.ds(s * ZROWS, ZROWS)])
        pltpu.sync_copy(z1, hd_spm.at[pl.ds(s * ZROWS, ZROWS)])
        pltpu.sync_copy(si.at[s], sbufs)
        pltpu.sync_copy(di.at[s], dbufs)
        plsc.subcore_barrier()

        @pl.loop(0, NCH_D)
        def _(i):
            pltpu.sync_copy(ones_v, hs_spm.at[sbufs.at[i]], add=True)
            pltpu.sync_copy(ones_v, hd_spm.at[dbufs.at[i]], add=True)

        plsc.subcore_barrier()
        pltpu.sync_copy(hs_spm.at[pl.ds(s * ZROWS, ZROWS)],
                        os_hbm.at[pl.ds(s * ZROWS, ZROWS)])
        pltpu.sync_copy(hd_spm.at[pl.ds(s * ZROWS, ZROWS)],
                        od_hbm.at[pl.ds(s * ZROWS, ZROWS)])
        plsc.subcore_barrier()

    @pl.when(c == 0)
    def _():
        do_graph(s1, d1, os1, od1)
        do_graph(s2, d2, os2, od2)

    @pl.when(c == 1)
    def _():
        do_graph(s3, d3, os3, od3)
        do_graph(s4, d4, os4, od4)


@functools.cache
def _deg_call():
    return functools.partial(
        pl.kernel,
        out_type=[jax.ShapeDtypeStruct((NPAD,), jnp.float32)] * 8,
        mesh=_sc_mesh(),
        scratch_types=[
            pltpu.VMEM((NCH_D, CH_D), jnp.int32),
            pltpu.VMEM((NCH_D, CH_D), jnp.int32),
            pltpu.VMEM((CH_D,), jnp.float32),
            pltpu.VMEM_SHARED((NPAD,), jnp.float32),
            pltpu.VMEM_SHARED((NPAD,), jnp.float32),
        ],
    )(_deg_body)


def _agg_body(z, h1, s1, d1, h2, s2, d2, h3, s3, d3, h4, s4, d4,
              o1, o2, o3, o4, sbA, dbA, sbB, dbB,
              rows0, rows1, rows2, rows3,
              gs0, gs1, gs2, gs3, ss0, ss1, ss2, ss3, aspm):
    c = lax.axis_index("c")
    s = lax.axis_index("s")
    rows = (rows0, rows1, rows2, rows3)
    gsem = (gs0, gs1, gs2, gs3)
    ssem = (ss0, ss1, ss2, ss3)

    def do_graph(h, si, di, o):
        def load_grp(g, sb, db):
            pltpu.sync_copy(si.at[s, pl.ds(g * GSZ, GSZ)], sb)
            pltpu.sync_copy(di.at[s, pl.ds(g * GSZ, GSZ)], db)

        def step(j, sb, db, sbn, dbn, wait_ss=True, cross=True):
            p = j % 4
            pltpu.make_async_copy(h.at[sb.at[j]], rows[p], gsem[p]).wait()
            pltpu.async_copy(rows[p], aspm.at[db.at[j]], ssem[p], add=True)
            q = (j + 2) % 4
            if wait_ss:
                pltpu.make_async_copy(rows[q], aspm.at[db.at[j - 2]],
                                      ssem[q]).wait()
            if cross:
                if j + 2 < GSZ:
                    pltpu.async_copy(h.at[sb.at[j + 2]], rows[q], gsem[q])
                else:
                    pltpu.async_copy(h.at[sbn.at[j - 6]], rows[q], gsem[q])

        load_grp(0, sbA, dbA)
        pltpu.async_copy(h.at[sbA.at[0]], rows[0], gsem[0])
        pltpu.async_copy(h.at[sbA.at[1]], rows[1], gsem[1])
        pltpu.sync_copy(z, aspm.at[pl.ds(s * ZROWS, ZROWS)])
        plsc.subcore_barrier()
        step(0, sbA, dbA, sbB, dbB, wait_ss=False)
        load_grp(1, sbB, dbB)
        step(1, sbA, dbA, sbB, dbB, wait_ss=False)
        for j in range(2, GSZ):
            step(j, sbA, dbA, sbB, dbB)

        @pl.loop(1, NGRP - 1, step=2)
        def _(g0):
            step(0, sbB, dbB, sbA, dbA)
            step(1, sbB, dbB, sbA, dbA)
            load_grp(g0 + 1, sbA, dbA)
            for j in range(2, GSZ):
                step(j, sbB, dbB, sbA, dbA)

            step(0, sbA, dbA, sbB, dbB)
            step(1, sbA, dbA, sbB, dbB)

            @pl.when(g0 + 2 < NGRP)
            def _():
                load_grp(g0 + 2, sbB, dbB)

            for j in range(2, GSZ - 2):
                step(j, sbA, dbA, sbB, dbB)
            for j in range(GSZ - 2, GSZ):
                step(j, sbA, dbA, sbB, dbB, cross=False)

                @pl.when(g0 + 2 < NGRP)
                def _():
                    q = (j + 2) % 4
                    pltpu.async_copy(h.at[sbB.at[j - 6]], rows[q], gsem[q])

        for j in range(GSZ - 2, GSZ):
            p = j % 4
            pltpu.make_async_copy(rows[p], aspm.at[dbA.at[j]],
                                  ssem[p]).wait()

        plsc.subcore_barrier()
        pltpu.sync_copy(aspm.at[pl.ds(s * ZROWS, ZROWS)],
                        o.at[pl.ds(s * ZROWS, ZROWS)])
        plsc.subcore_barrier()

    @pl.when(c == 0)
    def _():
        do_graph(h1, s1, d1, o1)
        do_graph(h2, s2, d2, o2)

    @pl.when(c == 1)
    def _():
        do_graph(h3, s3, d3, o3)
        do_graph(h4, s4, d4, o4)


@functools.cache
def _agg_call():
    return functools.partial(
        pl.kernel,
        out_type=[jax.ShapeDtypeStruct((NPAD, D), jnp.float32)] * 4,
        mesh=_sc_mesh(),
        scratch_types=[
        pltpu.VMEM((GSZ, CHUNK), jnp.int32),
        pltpu.VMEM((GSZ, CHUNK), jnp.int32),
        pltpu.VMEM((GSZ, CHUNK), jnp.int32),
        pltpu.VMEM((GSZ, CHUNK), jnp.int32),
        pltpu.VMEM((CHUNK, D), jnp.float32),
        pltpu.VMEM((CHUNK, D), jnp.float32),
        pltpu.VMEM((CHUNK, D), jnp.float32),
        pltpu.VMEM((CHUNK, D), jnp.float32),
        pltpu.SemaphoreType.DMA,
        pltpu.SemaphoreType.DMA,
        pltpu.SemaphoreType.DMA,
        pltpu.SemaphoreType.DMA,
        pltpu.SemaphoreType.DMA,
        pltpu.SemaphoreType.DMA,
        pltpu.SemaphoreType.DMA,
        pltpu.SemaphoreType.DMA,
        pltpu.VMEM_SHARED((NPAD, D), jnp.float32),
    ],
)(_agg_body)


def _prep_body(*refs):
    xs = refs[0:4]
    ods = refs[4:8]
    ids_ = refs[8:12]
    hs = refs[12:16]
    oss = refs[16:20]
    iss = refs[20:24]
    for g in range(4):
        osv = lax.rsqrt(jnp.maximum(ods[g][...], 1.0))
        isv = lax.rsqrt(jnp.maximum(ids_[g][...], 1.0))
        oss[g][...] = osv
        iss[g][...] = isv
        hs[g][...] = xs[g][...] * osv


def _prep(xp, od, idg):
    bx = pl.BlockSpec((BLK, D), lambda i: (i, 0))
    bs = pl.BlockSpec((BLK, 1), lambda i: (i, 0))
    return pl.pallas_call(
        _prep_body,
        grid=(GRID,),
        in_specs=[bx] * 4 + [bs] * 8,
        out_specs=[bx] * 4 + [bs] * 8,
        out_shape=[jax.ShapeDtypeStruct((NPAD, D), jnp.float32)] * 4
        + [jax.ShapeDtypeStruct((NPAD, 1), jnp.float32)] * 8,
    )(*xp, *od, *idg)


def _layer_body(w_ref, b_ref, *refs):
    aggs = refs[0:4]
    iss = refs[4:8]
    oss = refs[8:12]
    outs = refs[12:16]
    w = w_ref[...]
    b = b_ref[...]
    for g in range(4):
        a = aggs[g][...] * iss[g][...]
        h = jnp.maximum(jnp.dot(a, w, preferred_element_type=jnp.float32) + b,
                        0.0)
        outs[g][...] = h * oss[g][...]


def _layer(W, b, aggs, iss, oss):
    bx = pl.BlockSpec((BLK, D), lambda i: (i, 0))
    bs = pl.BlockSpec((BLK, 1), lambda i: (i, 0))
    bw = pl.BlockSpec((D, D), lambda i: (0, 0))
    bb = pl.BlockSpec((1, D), lambda i: (0, 0))
    return pl.pallas_call(
        _layer_body,
        grid=(GRID,),
        in_specs=[bw, bb] + [bx] * 4 + [bs] * 8,
        out_specs=[bx] * 4,
        out_shape=[jax.ShapeDtypeStruct((NPAD, D), jnp.float32)] * 4,
    )(W, b, *aggs, *iss, *oss)


def _fin_body(w_ref, b_ref, *refs):
    aggs = refs[0:4]
    iss = refs[4:8]
    out = refs[8]
    i = pl.program_id(0)

    @pl.when(i == 0)
    def _():
        out[...] = jnp.zeros((1, 1), jnp.float32)

    w = w_ref[...]
    b = b_ref[...]
    rowid = lax.broadcasted_iota(jnp.int32, (BLK, 1), 0) + i * BLK
    mask = rowid < N
    acc = jnp.float32(0.0)
    for g in range(4):
        a = aggs[g][...] * iss[g][...]
        h = jnp.maximum(jnp.dot(a, w, preferred_element_type=jnp.float32) + b,
                        0.0)
        acc += jnp.sum(jnp.where(mask, h, 0.0))
    out[...] = out[...] + jnp.reshape(acc, (1, 1))


def _fin(W, b, aggs, iss):
    bx = pl.BlockSpec((BLK, D), lambda i: (i, 0))
    bs = pl.BlockSpec((BLK, 1), lambda i: (i, 0))
    bw = pl.BlockSpec((D, D), lambda i: (0, 0))
    bb = pl.BlockSpec((1, D), lambda i: (0, 0))
    return pl.pallas_call(
        _fin_body,
        grid=(GRID,),
        in_specs=[bw, bb] + [bx] * 4 + [bs] * 4,
        out_specs=pl.BlockSpec((1, 1), lambda i: (0, 0)),
        out_shape=jax.ShapeDtypeStruct((1, 1), jnp.float32),
    )(W, b, *aggs, *iss)


def _prep_edges(g):
    g = g.astype(jnp.int32)
    pad = N + (jnp.arange(EPAD - E, dtype=jnp.int32) % TRASH)
    src = jnp.concatenate([g[0], pad]).reshape(NT, NCHUNK, CHUNK)
    dst = jnp.concatenate([g[1], pad]).reshape(NT, NCHUNK, CHUNK)
    return src, dst


def kernel(g1, x1, g2, x2, g3, x3, g4, x4, W1, b1, W2, b2, W3, b3):
    srcs, dsts = zip(*(_prep_edges(g) for g in (g1, g2, g3, g4)))

    def _degfmt(a):
        flat = a.reshape(NT, NCHUNK * CHUNK)
        extra = NCH_D * CH_D - NCHUNK * CHUNK
        pad = N + (jnp.arange(extra, dtype=jnp.int32) % TRASH)
        pad = jnp.broadcast_to(pad, (NT, extra))
        return jnp.concatenate([flat, pad], axis=1).reshape(NT, NCH_D, CH_D)

    sf = [_degfmt(a) for a in srcs]
    df = [_degfmt(a) for a in dsts]
    xp = [jnp.pad(x, ((0, TRASH), (0, 0))) for x in (x1, x2, x3, x4)]
    z = jnp.zeros((ZROWS, D), jnp.float32)
    z1 = jnp.zeros((ZROWS,), jnp.float32)

    degs = _deg_call()(z1, sf[0], df[0], sf[1], df[1], sf[2], df[2],
                       sf[3], df[3])
    od = [degs[2 * g][:, None] for g in range(4)]
    idg = [degs[2 * g + 1][:, None] for g in range(4)]

    prep = _prep(xp, od, idg)
    hs, oss, iss = prep[0:4], prep[4:8], prep[8:12]

    for W, b in ((W1, b1), (W2, b2)):
        aggs = _agg_call()(z, hs[0], srcs[0], dsts[0], hs[1], srcs[1],
                           dsts[1], hs[2], srcs[2], dsts[2], hs[3], srcs[3],
                           dsts[3])
        hs = _layer(W, b.reshape(1, D), aggs, iss, oss)

    aggs = _agg_call()(z, hs[0], srcs[0], dsts[0], hs[1], srcs[1], dsts[1],
                       hs[2], srcs[2], dsts[2], hs[3], srcs[3], dsts[3])
    total = _fin(W3, b3.reshape(1, D), aggs, iss)
    return total[0, 0] / jnp.float32(4 * N * D)


# speedup vs baseline: 1.0114x; 1.0107x over previous
import functools

import jax
import jax.numpy as jnp
from jax import lax
from jax.experimental import pallas as pl
from jax.experimental.pallas import tpu as pltpu
from jax.experimental.pallas import tpu_sc as plsc

N = 10000
E = 320000
D = 128
NPAD = 10240
TRASH = NPAD - N
NT = 16
CHUNK = 88
GSZ = 8
NCHUNK = 232
NGRP = NCHUNK // GSZ
EPAD = NT * NCHUNK * CHUNK
ZROWS = NPAD // NT
BLK = 1024
GRID = NPAD // BLK

@functools.cache
def _sc_mesh():
    return plsc.VectorSubcoreMesh(core_axis_name="c", subcore_axis_name="s",
                                  num_cores=2, num_subcores=NT)


CH_D = 128
NCH_D = 160


def _deg_body(z1, s1, d1, s2, d2, s3, d3, s4, d4,
              os1, od1, os2, od2, os3, od3, os4, od4,
              sbufs, dbufs, ones_v, dsa, dsb, hs_spm, hd_spm):
    c = lax.axis_index("c")
    s = lax.axis_index("s")
    for j in range(CH_D // 16):
        ones_v[pl.ds(j * 16, 16)] = jnp.ones((16,), jnp.float32)

    def do_graph(si, di, os_hbm, od_hbm):
        pltpu.sync_copy(z1, hs_spm.at[pl.ds(s * ZROWS, ZROWS)])
        pltpu.sync_copy(z1, hd_spm.at[pl.ds(s * ZROWS, ZROWS)])
        pltpu.sync_copy(si.at[s], sbufs)
        pltpu.sync_copy(di.at[s], dbufs)
        plsc.subcore_barrier()

        @pl.loop(0, NCH_D)
        def _(i):
            pltpu.async_copy(ones_v, hs_spm.at[sbufs.at[i]], dsa, add=True)
            pltpu.async_copy(ones_v, hd_spm.at[dbufs.at[i]], dsb, add=True)
            pltpu.make_async_copy(ones_v, hs_spm.at[sbufs.at[i]], dsa).wait()
            pltpu.make_async_copy(ones_v, hd_spm.at[dbufs.at[i]], dsb).wait()

        plsc.subcore_barrier()
        pltpu.sync_copy(hs_spm.at[pl.ds(s * ZROWS, ZROWS)],
                        os_hbm.at[pl.ds(s * ZROWS, ZROWS)])
        pltpu.sync_copy(hd_spm.at[pl.ds(s * ZROWS, ZROWS)],
                        od_hbm.at[pl.ds(s * ZROWS, ZROWS)])
        plsc.subcore_barrier()

    @pl.when(c == 0)
    def _():
        do_graph(s1, d1, os1, od1)
        do_graph(s2, d2, os2, od2)

    @pl.when(c == 1)
    def _():
        do_graph(s3, d3, os3, od3)
        do_graph(s4, d4, os4, od4)


@functools.cache
def _deg_call():
    return functools.partial(
        pl.kernel,
        out_type=[jax.ShapeDtypeStruct((NPAD,), jnp.float32)] * 8,
        mesh=_sc_mesh(),
        scratch_types=[
            pltpu.VMEM((NCH_D, CH_D), jnp.int32),
            pltpu.VMEM((NCH_D, CH_D), jnp.int32),
            pltpu.VMEM((CH_D,), jnp.float32),
            pltpu.SemaphoreType.DMA,
            pltpu.SemaphoreType.DMA,
            pltpu.VMEM_SHARED((NPAD,), jnp.float32),
            pltpu.VMEM_SHARED((NPAD,), jnp.float32),
        ],
    )(_deg_body)


def _agg_body(z, h1, s1, d1, h2, s2, d2, h3, s3, d3, h4, s4, d4,
              o1, o2, o3, o4, sbA, dbA, sbB, dbB,
              rows0, rows1, rows2, rows3,
              gs0, gs1, gs2, gs3, ss0, ss1, ss2, ss3, aspm):
    c = lax.axis_index("c")
    s = lax.axis_index("s")
    rows = (rows0, rows1, rows2, rows3)
    gsem = (gs0, gs1, gs2, gs3)
    ssem = (ss0, ss1, ss2, ss3)

    def do_graph(h, si, di, o):
        def load_grp(g, sb, db):
            pltpu.sync_copy(si.at[s, pl.ds(g * GSZ, GSZ)], sb)
            pltpu.sync_copy(di.at[s, pl.ds(g * GSZ, GSZ)], db)

        def step(j, sb, db, sbn, dbn, wait_ss=True, cross=True):
            p = j % 4
            pltpu.make_async_copy(h.at[sb.at[j]], rows[p], gsem[p]).wait()
            pltpu.async_copy(rows[p], aspm.at[db.at[j]], ssem[p], add=True)
            q = (j + 2) % 4
            if wait_ss:
                pltpu.make_async_copy(rows[q], aspm.at[db.at[j - 2]],
                                      ssem[q]).wait()
            if cross:
                if j + 2 < GSZ:
                    pltpu.async_copy(h.at[sb.at[j + 2]], rows[q], gsem[q])
                else:
                    pltpu.async_copy(h.at[sbn.at[j - 6]], rows[q], gsem[q])

        load_grp(0, sbA, dbA)
        pltpu.async_copy(h.at[sbA.at[0]], rows[0], gsem[0])
        pltpu.async_copy(h.at[sbA.at[1]], rows[1], gsem[1])
        pltpu.sync_copy(z, aspm.at[pl.ds(s * ZROWS, ZROWS)])
        plsc.subcore_barrier()
        step(0, sbA, dbA, sbB, dbB, wait_ss=False)
        load_grp(1, sbB, dbB)
        step(1, sbA, dbA, sbB, dbB, wait_ss=False)
        for j in range(2, GSZ):
            step(j, sbA, dbA, sbB, dbB)

        @pl.loop(1, NGRP - 1, step=2)
        def _(g0):
            step(0, sbB, dbB, sbA, dbA)
            step(1, sbB, dbB, sbA, dbA)
            load_grp(g0 + 1, sbA, dbA)
            for j in range(2, GSZ):
                step(j, sbB, dbB, sbA, dbA)

            step(0, sbA, dbA, sbB, dbB)
            step(1, sbA, dbA, sbB, dbB)

            @pl.when(g0 + 2 < NGRP)
            def _():
                load_grp(g0 + 2, sbB, dbB)

            for j in range(2, GSZ - 2):
                step(j, sbA, dbA, sbB, dbB)
            for j in range(GSZ - 2, GSZ):
                step(j, sbA, dbA, sbB, dbB, cross=False)

                @pl.when(g0 + 2 < NGRP)
                def _():
                    q = (j + 2) % 4
                    pltpu.async_copy(h.at[sbB.at[j - 6]], rows[q], gsem[q])

        for j in range(GSZ - 2, GSZ):
            p = j % 4
            pltpu.make_async_copy(rows[p], aspm.at[dbA.at[j]],
                                  ssem[p]).wait()

        plsc.subcore_barrier()
        pltpu.sync_copy(aspm.at[pl.ds(s * ZROWS, ZROWS)],
                        o.at[pl.ds(s * ZROWS, ZROWS)])
        plsc.subcore_barrier()

    @pl.when(c == 0)
    def _():
        do_graph(h1, s1, d1, o1)
        do_graph(h2, s2, d2, o2)

    @pl.when(c == 1)
    def _():
        do_graph(h3, s3, d3, o3)
        do_graph(h4, s4, d4, o4)


@functools.cache
def _agg_call():
    return functools.partial(
        pl.kernel,
        out_type=[jax.ShapeDtypeStruct((NPAD, D), jnp.float32)] * 4,
        mesh=_sc_mesh(),
        scratch_types=[
        pltpu.VMEM((GSZ, CHUNK), jnp.int32),
        pltpu.VMEM((GSZ, CHUNK), jnp.int32),
        pltpu.VMEM((GSZ, CHUNK), jnp.int32),
        pltpu.VMEM((GSZ, CHUNK), jnp.int32),
        pltpu.VMEM((CHUNK, D), jnp.float32),
        pltpu.VMEM((CHUNK, D), jnp.float32),
        pltpu.VMEM((CHUNK, D), jnp.float32),
        pltpu.VMEM((CHUNK, D), jnp.float32),
        pltpu.SemaphoreType.DMA,
        pltpu.SemaphoreType.DMA,
        pltpu.SemaphoreType.DMA,
        pltpu.SemaphoreType.DMA,
        pltpu.SemaphoreType.DMA,
        pltpu.SemaphoreType.DMA,
        pltpu.SemaphoreType.DMA,
        pltpu.SemaphoreType.DMA,
        pltpu.VMEM_SHARED((NPAD, D), jnp.float32),
    ],
)(_agg_body)


def _prep_body(*refs):
    xs = refs[0:4]
    ods = refs[4:8]
    ids_ = refs[8:12]
    hs = refs[12:16]
    oss = refs[16:20]
    iss = refs[20:24]
    for g in range(4):
        osv = lax.rsqrt(jnp.maximum(ods[g][...], 1.0))
        isv = lax.rsqrt(jnp.maximum(ids_[g][...], 1.0))
        oss[g][...] = osv
        iss[g][...] = isv
        hs[g][...] = xs[g][...] * osv


def _prep(xp, od, idg):
    bx = pl.BlockSpec((BLK, D), lambda i: (i, 0))
    bs = pl.BlockSpec((BLK, 1), lambda i: (i, 0))
    return pl.pallas_call(
        _prep_body,
        grid=(GRID,),
        in_specs=[bx] * 4 + [bs] * 8,
        out_specs=[bx] * 4 + [bs] * 8,
        out_shape=[jax.ShapeDtypeStruct((NPAD, D), jnp.float32)] * 4
        + [jax.ShapeDtypeStruct((NPAD, 1), jnp.float32)] * 8,
    )(*xp, *od, *idg)


def _layer_body(w_ref, b_ref, *refs):
    aggs = refs[0:4]
    iss = refs[4:8]
    oss = refs[8:12]
    outs = refs[12:16]
    w = w_ref[...]
    b = b_ref[...]
    for g in range(4):
        a = aggs[g][...] * iss[g][...]
        h = jnp.maximum(jnp.dot(a, w, preferred_element_type=jnp.float32) + b,
                        0.0)
        outs[g][...] = h * oss[g][...]


def _layer(W, b, aggs, iss, oss):
    bx = pl.BlockSpec((BLK, D), lambda i: (i, 0))
    bs = pl.BlockSpec((BLK, 1), lambda i: (i, 0))
    bw = pl.BlockSpec((D, D), lambda i: (0, 0))
    bb = pl.BlockSpec((1, D), lambda i: (0, 0))
    return pl.pallas_call(
        _layer_body,
        grid=(GRID,),
        in_specs=[bw, bb] + [bx] * 4 + [bs] * 8,
        out_specs=[bx] * 4,
        out_shape=[jax.ShapeDtypeStruct((NPAD, D), jnp.float32)] * 4,
    )(W, b, *aggs, *iss, *oss)


def _fin_body(w_ref, b_ref, *refs):
    aggs = refs[0:4]
    iss = refs[4:8]
    out = refs[8]
    i = pl.program_id(0)

    @pl.when(i == 0)
    def _():
        out[...] = jnp.zeros((1, 1), jnp.float32)

    w = w_ref[...]
    b = b_ref[...]
    rowid = lax.broadcasted_iota(jnp.int32, (BLK, 1), 0) + i * BLK
    mask = rowid < N
    acc = jnp.float32(0.0)
    for g in range(4):
        a = aggs[g][...] * iss[g][...]
        h = jnp.maximum(jnp.dot(a, w, preferred_element_type=jnp.float32) + b,
                        0.0)
        acc += jnp.sum(jnp.where(mask, h, 0.0))
    out[...] = out[...] + jnp.reshape(acc, (1, 1))


def _fin(W, b, aggs, iss):
    bx = pl.BlockSpec((BLK, D), lambda i: (i, 0))
    bs = pl.BlockSpec((BLK, 1), lambda i: (i, 0))
    bw = pl.BlockSpec((D, D), lambda i: (0, 0))
    bb = pl.BlockSpec((1, D), lambda i: (0, 0))
    return pl.pallas_call(
        _fin_body,
        grid=(GRID,),
        in_specs=[bw, bb] + [bx] * 4 + [bs] * 4,
        out_specs=pl.BlockSpec((1, 1), lambda i: (0, 0)),
        out_shape=jax.ShapeDtypeStruct((1, 1), jnp.float32),
    )(W, b, *aggs, *iss)


def _prep_edges(g):
    g = g.astype(jnp.int32)
    pad = N + (jnp.arange(EPAD - E, dtype=jnp.int32) % TRASH)
    src = jnp.concatenate([g[0], pad]).reshape(NT, NCHUNK, CHUNK)
    dst = jnp.concatenate([g[1], pad]).reshape(NT, NCHUNK, CHUNK)
    return src, dst


def kernel(g1, x1, g2, x2, g3, x3, g4, x4, W1, b1, W2, b2, W3, b3):
    srcs, dsts = zip(*(_prep_edges(g) for g in (g1, g2, g3, g4)))

    def _degfmt(a):
        flat = a.reshape(NT, NCHUNK * CHUNK)
        extra = NCH_D * CH_D - NCHUNK * CHUNK
        pad = N + (jnp.arange(extra, dtype=jnp.int32) % TRASH)
        pad = jnp.broadcast_to(pad, (NT, extra))
        return jnp.concatenate([flat, pad], axis=1).reshape(NT, NCH_D, CH_D)

    sf = [_degfmt(a) for a in srcs]
    df = [_degfmt(a) for a in dsts]
    xp = [jnp.pad(x, ((0, TRASH), (0, 0))) for x in (x1, x2, x3, x4)]
    z = jnp.zeros((ZROWS, D), jnp.float32)
    z1 = jnp.zeros((ZROWS,), jnp.float32)

    degs = _deg_call()(z1, sf[0], df[0], sf[1], df[1], sf[2], df[2],
                       sf[3], df[3])
    od = [degs[2 * g][:, None] for g in range(4)]
    idg = [degs[2 * g + 1][:, None] for g in range(4)]

    prep = _prep(xp, od, idg)
    hs, oss, iss = prep[0:4], prep[4:8], prep[8:12]

    for W, b in ((W1, b1), (W2, b2)):
        aggs = _agg_call()(z, hs[0], srcs[0], dsts[0], hs[1], srcs[1],
                           dsts[1], hs[2], srcs[2], dsts[2], hs[3], srcs[3],
                           dsts[3])
        hs = _layer(W, b.reshape(1, D), aggs, iss, oss)

    aggs = _agg_call()(z, hs[0], srcs[0], dsts[0], hs[1], srcs[1], dsts[1],
                       hs[2], srcs[2], dsts[2], hs[3], srcs[3], dsts[3])
    total = _fin(W3, b3.reshape(1, D), aggs, iss)
    return total[0, 0] / jnp.float32(4 * N * D)
